# Initial kernel scaffold; baseline (speedup 1.0000x reference)
import functools

import jax
import jax.numpy as jnp
from jax import lax
from jax.experimental import pallas as pl
from jax.experimental.pallas import tpu as pltpu
from jax.experimental.pallas import tpu_sc as plsc

N_NODES = 10000
N_EDGES = 320000
H = 128
L = 16
CHUNK = 128
N_CHUNKS = N_EDGES // CHUNK
NW = 32
ROWS_PER_TILE = N_NODES // 16
ZROWS = 125

_sc_mesh = plsc.VectorSubcoreMesh(core_axis_name="c", subcore_axis_name="s")


@functools.partial(
    pl.kernel,
    out_type=jax.ShapeDtypeStruct((2, N_NODES, H), jnp.float32),
    mesh=_sc_mesh,
    scratch_types=[
        pltpu.VMEM((CHUNK,), jnp.int32),
        pltpu.VMEM((CHUNK,), jnp.int32),
        pltpu.VMEM((CHUNK, H), jnp.float32),
        pltpu.VMEM((CHUNK, H), jnp.float32),
        pltpu.VMEM((ZROWS, H), jnp.float32),
        pltpu.VMEM_SHARED((N_NODES, H), jnp.float32),
        pltpu.SemaphoreType.DMA,
        pltpu.SemaphoreType.DMA,
    ],
)
def _sc_aggregate(x_hbm, src_hbm, dst_hbm, ea_hbm, out_hbm,
                  src_v, dst_v, xr_v, ea_v, zbuf_v, acc_sh, gsem, esem):
    c = lax.axis_index("c")
    s = lax.axis_index("s")
    wid = s * 2 + c

    zeros16 = jnp.zeros((L,), jnp.float32)

    def zrow(r, _):
        for j in range(H // L):
            zbuf_v[r, pl.ds(j * L, L)] = zeros16
        return 0

    lax.fori_loop(0, ZROWS, zrow, 0)
    slab = s * ROWS_PER_TILE
    for i in range(ROWS_PER_TILE // ZROWS):
        pltpu.sync_copy(zbuf_v, acc_sh.at[pl.ds(slab + i * ZROWS, ZROWS), :])
    plsc.subcore_barrier()

    n_my = 78 + jnp.where(wid < N_CHUNKS - 78 * NW, 1, 0)

    def chunk_body(k, _):
        base = (wid + k * NW) * CHUNK
        pltpu.sync_copy(src_hbm.at[pl.ds(base, CHUNK)], src_v)
        pltpu.sync_copy(dst_hbm.at[pl.ds(base, CHUNK)], dst_v)
        gcp = pltpu.async_copy(x_hbm.at[src_v], xr_v, gsem)
        ecp = pltpu.async_copy(ea_hbm.at[pl.ds(base, CHUNK), :], ea_v, esem)
        gcp.wait()
        ecp.wait()

        def row(r, _):
            for j in range(H // L):
                sl = pl.ds(j * L, L)
                ea_v[r, sl] = jnp.maximum(xr_v[r, sl] + ea_v[r, sl], 0.0)
            return 0

        lax.fori_loop(0, CHUNK, row, 0)
        pltpu.sync_copy(ea_v, acc_sh.at[dst_v], add=True)
        return 0

    lax.fori_loop(0, n_my, chunk_body, 0)
    plsc.subcore_barrier()

    pltpu.sync_copy(acc_sh.at[pl.ds(slab, ROWS_PER_TILE), :],
                    out_hbm.at[c, pl.ds(slab, ROWS_PER_TILE), :])


def _tc_layer_body(x_ref, p0_ref, p1_ref, wt_ref, b_ref, o_ref):
    h = x_ref[...] + p0_ref[...] + p1_ref[...]
    y = jnp.dot(h, wt_ref[...], preferred_element_type=jnp.float32) + b_ref[...]
    o_ref[...] = jnp.maximum(y, 0.0)


_TC_BLOCK = 1000


def _tc_layer(x, p0, p1, wt, b2d):
    grid = (N_NODES // _TC_BLOCK,)
    return pl.pallas_call(
        _tc_layer_body,
        grid=grid,
        in_specs=[
            pl.BlockSpec((_TC_BLOCK, H), lambda i: (i, 0)),
            pl.BlockSpec((_TC_BLOCK, H), lambda i: (i, 0)),
            pl.BlockSpec((_TC_BLOCK, H), lambda i: (i, 0)),
            pl.BlockSpec((H, H), lambda i: (0, 0)),
            pl.BlockSpec((1, H), lambda i: (0, 0)),
        ],
        out_specs=pl.BlockSpec((_TC_BLOCK, H), lambda i: (i, 0)),
        out_shape=jax.ShapeDtypeStruct((N_NODES, H), jnp.float32),
    )(x, p0, p1, wt, b2d)


def kernel(node_feats, edge_index, edge_attrs, W1, b1, W2, b2):
    src = edge_index[0].astype(jnp.int32)
    dst = edge_index[1].astype(jnp.int32)
    x = node_feats

    p = _sc_aggregate(x, src, dst, edge_attrs)
    x1 = _tc_layer(x, p[0], p[1], W1.T, b1.reshape(1, H))
    q = _sc_aggregate(x1, src, dst, edge_attrs)
    x2 = _tc_layer(x1, q[0], q[1], W2.T, b2.reshape(1, H))
    return x2


# speedup vs baseline: 4.1682x; 4.1682x over previous
import functools

import jax
import jax.numpy as jnp
from jax import lax
from jax.experimental import pallas as pl
from jax.experimental.pallas import tpu as pltpu
from jax.experimental.pallas import tpu_sc as plsc

N_NODES = 10000
N_EDGES = 320000
H = 128
L = 16
CHUNK = 128
N_CHUNKS = N_EDGES // CHUNK
NW = 32
SLAB = 624
ZROWS = 104

_sc_mesh = plsc.VectorSubcoreMesh(core_axis_name="c", subcore_axis_name="s")


@functools.partial(
    pl.kernel,
    out_type=jax.ShapeDtypeStruct((2 * N_NODES, H), jnp.float32),
    mesh=_sc_mesh,
    scratch_types=[
        pltpu.VMEM((CHUNK,), jnp.int32),
        pltpu.VMEM((CHUNK,), jnp.int32),
        pltpu.VMEM((CHUNK, H), jnp.float32),
        pltpu.VMEM((CHUNK, H), jnp.float32),
        pltpu.VMEM((ZROWS, H), jnp.float32),
        pltpu.VMEM_SHARED((N_NODES, H), jnp.float32),
        pltpu.SemaphoreType.DMA,
        pltpu.SemaphoreType.DMA,
    ],
)
def _sc_aggregate(x_hbm, src_hbm, dst_hbm, ea_hbm, out_hbm,
                  src_v, dst_v, xr_v, ea_v, zbuf_v, acc_sh, gsem, esem):
    c = lax.axis_index("c")
    s = lax.axis_index("s")
    wid = s * 2 + c

    zeros16 = jnp.zeros((L,), jnp.float32)

    def zrow(r, _):
        for j in range(H // L):
            zbuf_v[r, pl.ds(j * L, L)] = zeros16
        return 0

    lax.fori_loop(0, ZROWS, zrow, 0)
    slab = s * SLAB
    for i in range(SLAB // ZROWS):
        pltpu.sync_copy(zbuf_v, acc_sh.at[pl.ds(slab + i * ZROWS, ZROWS), :])

    @pl.when(s == 15)
    def _():
        pltpu.sync_copy(zbuf_v.at[pl.ds(0, 16), :],
                        acc_sh.at[pl.ds(16 * SLAB, 16), :])

    plsc.subcore_barrier()

    n_my = 78 + jnp.where(wid < N_CHUNKS - 78 * NW, 1, 0)

    def chunk_body(k, _):
        base = (wid + k * NW) * CHUNK
        pltpu.sync_copy(src_hbm.at[pl.ds(base, CHUNK)], src_v)
        pltpu.sync_copy(dst_hbm.at[pl.ds(base, CHUNK)], dst_v)
        gcp = pltpu.async_copy(x_hbm.at[src_v], xr_v, gsem)
        ecp = pltpu.async_copy(ea_hbm.at[pl.ds(base, CHUNK), :], ea_v, esem)
        gcp.wait()
        ecp.wait()

        def row(r, _):
            for j in range(H // L):
                sl = pl.ds(j * L, L)
                ea_v[r, sl] = jnp.maximum(xr_v[r, sl] + ea_v[r, sl], 0.0)
            return 0

        lax.fori_loop(0, CHUNK, row, 0)
        pltpu.sync_copy(ea_v, acc_sh.at[dst_v], add=True)
        return 0

    lax.fori_loop(0, n_my, chunk_body, 0)
    plsc.subcore_barrier()

    obase = c * N_NODES + slab
    for i in range(SLAB // ZROWS):
        pltpu.sync_copy(acc_sh.at[pl.ds(slab + i * ZROWS, ZROWS), :],
                        out_hbm.at[pl.ds(obase + i * ZROWS, ZROWS), :])

    @pl.when(s == 15)
    def _():
        pltpu.sync_copy(acc_sh.at[pl.ds(16 * SLAB, 16), :],
                        out_hbm.at[pl.ds(c * N_NODES + 16 * SLAB, 16), :])


def _tc_layer_body(x_ref, p0_ref, p1_ref, wt_ref, b_ref, o_ref):
    h = x_ref[...] + p0_ref[...] + p1_ref[...]
    y = jnp.dot(h, wt_ref[...], preferred_element_type=jnp.float32) + b_ref[...]
    o_ref[...] = jnp.maximum(y, 0.0)


_TC_BLOCK = 1000


def _tc_layer(x, p0, p1, wt, b2d):
    grid = (N_NODES // _TC_BLOCK,)
    return pl.pallas_call(
        _tc_layer_body,
        grid=grid,
        in_specs=[
            pl.BlockSpec((_TC_BLOCK, H), lambda i: (i, 0)),
            pl.BlockSpec((_TC_BLOCK, H), lambda i: (i, 0)),
            pl.BlockSpec((_TC_BLOCK, H), lambda i: (i, 0)),
            pl.BlockSpec((H, H), lambda i: (0, 0)),
            pl.BlockSpec((1, H), lambda i: (0, 0)),
        ],
        out_specs=pl.BlockSpec((_TC_BLOCK, H), lambda i: (i, 0)),
        out_shape=jax.ShapeDtypeStruct((N_NODES, H), jnp.float32),
    )(x, p0, p1, wt, b2d)


def kernel(node_feats, edge_index, edge_attrs, W1, b1, W2, b2):
    src = edge_index[0].astype(jnp.int32)
    dst = edge_index[1].astype(jnp.int32)
    x = node_feats

    p = _sc_aggregate(x, src, dst, edge_attrs)
    x1 = _tc_layer(x, p[:N_NODES], p[N_NODES:], W1.T, b1.reshape(1, H))
    q = _sc_aggregate(x1, src, dst, edge_attrs)
    x2 = _tc_layer(x1, q[:N_NODES], q[N_NODES:], W2.T, b2.reshape(1, H))
    return x2


# speedup vs baseline: 7.1256x; 1.7095x over previous
import functools

import jax
import jax.numpy as jnp
from jax import lax
from jax.experimental import pallas as pl
from jax.experimental.pallas import tpu as pltpu
from jax.experimental.pallas import tpu_sc as plsc

N_NODES = 10000
N_EDGES = 320000
H = 128
L = 16
CHUNK = 64
N_CHUNKS = N_EDGES // CHUNK
NW = 32
BASE_CHUNKS = N_CHUNKS // NW
EXTRA = N_CHUNKS - BASE_CHUNKS * NW
SLAB = 624

_sc_mesh = plsc.VectorSubcoreMesh(core_axis_name="c", subcore_axis_name="s")


@functools.partial(
    pl.kernel,
    out_type=jax.ShapeDtypeStruct((2 * N_NODES, H), jnp.float32),
    mesh=_sc_mesh,
    scratch_types=[
        [pltpu.VMEM((CHUNK,), jnp.int32) for _ in range(3)],
        [pltpu.VMEM((CHUNK,), jnp.int32) for _ in range(3)],
        [pltpu.VMEM((CHUNK, H), jnp.float32) for _ in range(3)],
        [pltpu.VMEM((CHUNK, H), jnp.float32) for _ in range(3)],
        pltpu.VMEM_SHARED((N_NODES, H), jnp.float32),
        [pltpu.SemaphoreType.DMA for _ in range(3)],
        [pltpu.SemaphoreType.DMA for _ in range(3)],
        [pltpu.SemaphoreType.DMA for _ in range(3)],
        [pltpu.SemaphoreType.DMA for _ in range(3)],
    ],
)
def _sc_aggregate(x_hbm, src_hbm, dst_hbm, ea_hbm, out_hbm,
                  svs, dvs, xrs, eas, acc_sh, gss, ess, iss, sss):
    c = lax.axis_index("c")
    s = lax.axis_index("s")
    wid = s * 2 + c

    zeros16 = jnp.zeros((L,), jnp.float32)
    zb = eas[0]

    def zrow(r, _):
        for j in range(H // L):
            zb[r, pl.ds(j * L, L)] = zeros16
        return 0

    lax.fori_loop(0, CHUNK, zrow, 0)
    slab = s * SLAB
    for i in range(SLAB // CHUNK):
        pltpu.sync_copy(zb, acc_sh.at[pl.ds(slab + i * CHUNK, CHUNK), :])
    pltpu.sync_copy(zb.at[pl.ds(0, SLAB % CHUNK), :],
                    acc_sh.at[pl.ds(slab + (SLAB // CHUNK) * CHUNK,
                                    SLAB % CHUNK), :])

    @pl.when(s == 15)
    def _():
        pltpu.sync_copy(zb.at[pl.ds(0, 16), :],
                        acc_sh.at[pl.ds(16 * SLAB, 16), :])

    plsc.subcore_barrier()

    n_my = BASE_CHUNKS + jnp.where(wid < EXTRA, 1, 0)
    ebase = (BASE_CHUNKS * wid + jnp.minimum(wid, EXTRA)) * CHUNK

    def idx_start(kk, b):
        off = ebase + kk * CHUNK
        pltpu.async_copy(src_hbm.at[pl.ds(off, CHUNK)], svs[b], iss[b])
        pltpu.async_copy(dst_hbm.at[pl.ds(off, CHUNK)], dvs[b], iss[b])

    def idx_wait(b):
        pltpu.make_async_copy(src_hbm.at[pl.ds(0, CHUNK)], svs[b],
                              iss[b]).wait()
        pltpu.make_async_copy(dst_hbm.at[pl.ds(0, CHUNK)], dvs[b],
                              iss[b]).wait()

    def data_start(kk, b):
        pltpu.async_copy(x_hbm.at[svs[b]], xrs[b], gss[b])
        pltpu.async_copy(ea_hbm.at[pl.ds(ebase + kk * CHUNK, CHUNK), :],
                         eas[b], ess[b])

    def data_wait(b):
        pltpu.make_async_copy(x_hbm.at[svs[b]], xrs[b], gss[b]).wait()
        pltpu.make_async_copy(ea_hbm.at[pl.ds(0, CHUNK), :], eas[b],
                              ess[b]).wait()

    def compute(b):
        xr, ea = xrs[b], eas[b]

        @plsc.parallel_loop(0, CHUNK, unroll=4)
        def _(r):
            for j in range(H // L):
                sl = pl.ds(j * L, L)
                ea[r, sl] = jnp.maximum(xr[r, sl] + ea[r, sl], 0.0)

    def scatter_start(b):
        pltpu.async_copy(eas[b], acc_sh.at[dvs[b]], sss[b], add=True)

    def scatter_wait(b):
        pltpu.make_async_copy(eas[b], acc_sh.at[dvs[b]], sss[b]).wait()

    idx_start(0, 0)
    idx_wait(0)
    data_start(0, 0)
    idx_start(1, 1)

    @pl.loop(0, BASE_CHUNKS, step=3)
    def _(k):
        for b in range(3):
            kk = k + b
            bn = (b + 1) % 3
            bp = (b + 2) % 3

            @pl.when(kk + 1 < n_my)
            def _():
                idx_wait(bn)
                data_start(kk + 1, bn)

            data_wait(b)
            compute(b)

            @pl.when(kk > 0)
            def _():
                scatter_wait(bp)

            @pl.when(kk + 2 < n_my)
            def _():
                idx_start(kk + 2, bp)

            scatter_start(b)

    @pl.when(wid < EXTRA)
    def _():
        data_wait(0)
        compute(0)
        scatter_wait(2)
        scatter_start(0)
        scatter_wait(0)

    @pl.when(wid >= EXTRA)
    def _():
        scatter_wait(2)

    plsc.subcore_barrier()

    obase = c * N_NODES + slab
    for i in range(SLAB // CHUNK):
        pltpu.sync_copy(acc_sh.at[pl.ds(slab + i * CHUNK, CHUNK), :],
                        out_hbm.at[pl.ds(obase + i * CHUNK, CHUNK), :])
    pltpu.sync_copy(
        acc_sh.at[pl.ds(slab + (SLAB // CHUNK) * CHUNK, SLAB % CHUNK), :],
        out_hbm.at[pl.ds(obase + (SLAB // CHUNK) * CHUNK, SLAB % CHUNK), :])

    @pl.when(s == 15)
    def _():
        pltpu.sync_copy(acc_sh.at[pl.ds(16 * SLAB, 16), :],
                        out_hbm.at[pl.ds(c * N_NODES + 16 * SLAB, 16), :])


def _tc_layer_body(x_ref, p0_ref, p1_ref, wt_ref, b_ref, o_ref):
    h = x_ref[...] + p0_ref[...] + p1_ref[...]
    y = jnp.dot(h, wt_ref[...], preferred_element_type=jnp.float32) + b_ref[...]
    o_ref[...] = jnp.maximum(y, 0.0)


_TC_BLOCK = 1000


def _tc_layer(x, p0, p1, wt, b2d):
    grid = (N_NODES // _TC_BLOCK,)
    return pl.pallas_call(
        _tc_layer_body,
        grid=grid,
        in_specs=[
            pl.BlockSpec((_TC_BLOCK, H), lambda i: (i, 0)),
            pl.BlockSpec((_TC_BLOCK, H), lambda i: (i, 0)),
            pl.BlockSpec((_TC_BLOCK, H), lambda i: (i, 0)),
            pl.BlockSpec((H, H), lambda i: (0, 0)),
            pl.BlockSpec((1, H), lambda i: (0, 0)),
        ],
        out_specs=pl.BlockSpec((_TC_BLOCK, H), lambda i: (i, 0)),
        out_shape=jax.ShapeDtypeStruct((N_NODES, H), jnp.float32),
    )(x, p0, p1, wt, b2d)


def kernel(node_feats, edge_index, edge_attrs, W1, b1, W2, b2):
    src = edge_index[0].astype(jnp.int32)
    dst = edge_index[1].astype(jnp.int32)
    x = node_feats

    p = _sc_aggregate(x, src, dst, edge_attrs)
    x1 = _tc_layer(x, p[:N_NODES], p[N_NODES:], W1.T, b1.reshape(1, H))
    q = _sc_aggregate(x1, src, dst, edge_attrs)
    x2 = _tc_layer(x1, q[:N_NODES], q[N_NODES:], W2.T, b2.reshape(1, H))
    return x2
